# baseline (speedup 1.0000x reference)
import functools

import jax
import jax.numpy as jnp
from jax import lax
from jax.experimental import pallas as pl
from jax.experimental.pallas import tpu as pltpu
from jax.experimental.pallas import tpu_sc as plsc

_NW = 32


@functools.lru_cache(maxsize=None)
def _make_gather(V, D, R, C):
    B = R * C
    bpw = B // _NW
    assert C % bpw == 0, (R, C, bpw)
    chunk = bpw
    while chunk * (D + 1) * 4 > 460000:
        chunk //= 2
    steps = bpw // chunk
    out_shape = (R, C, D) if R > 1 else (C, D)
    mesh = plsc.VectorSubcoreMesh(core_axis_name="c", subcore_axis_name="s")

    @functools.partial(
        pl.kernel, mesh=mesh,
        out_type=jax.ShapeDtypeStruct(out_shape, jnp.float32),
        scratch_types=[pltpu.VMEM((chunk,), jnp.int32),
                       pltpu.VMEM((chunk, D), jnp.float32),
                       pltpu.SemaphoreType.DMA],
        compiler_params=pltpu.CompilerParams(use_tc_tiling_on_sc=False),
    )
    def gk(table_hbm, idx_hbm, out_hbm, idx_v, rows_v, sem):
        wid = lax.axis_index("s") * 2 + lax.axis_index("c")
        base = wid * bpw
        slab = base // C
        col0 = base % C
        for i in range(steps):
            off = base + i * chunk
            pltpu.sync_copy(idx_hbm.at[pl.ds(off, chunk)], idx_v)
            pltpu.async_copy(table_hbm.at[idx_v], rows_v, sem).wait()
            if R > 1:
                dst = out_hbm.at[slab, pl.ds(col0 + i * chunk, chunk)]
            else:
                dst = out_hbm.at[pl.ds(off, chunk)]
            pltpu.sync_copy(rows_v, dst)

    return gk


def _gather_rows(table, idx, R=1):
    V, D = table.shape
    (B,) = idx.shape
    return _make_gather(V, D, R, B // R)(table, idx)


def _wspec(shape):
    nd = len(shape)
    return pl.BlockSpec(shape, lambda i, _n=nd: (0,) * _n)


def _rspec(nb, c):
    return pl.BlockSpec((nb, c), lambda i: (i, 0))


def _kspec(k, nb, c):
    return pl.BlockSpec((k, nb, c), lambda i: (0, i, 0))


def _leaky(x):
    return jnp.where(x >= 0, x, 0.2 * x)


def _fold(p):
    return p["W"] * p["g"][None, :], (p["b"] * p["g"] + p["be"])[None, :]


def _k1_body(x_ref, xyz_ref, W0, b0, W1, b1, f0_ref, t0_ref):
    y = x_ref[...] @ W0[...] + b0[...]
    f0 = _leaky(y)
    ft = jnp.maximum(f0 @ W1[...] + b1[...], 0.0)
    f0_ref[...] = f0
    nb = ft.shape[0]
    t0_ref[...] = jnp.concatenate(
        [ft, xyz_ref[...], jnp.zeros((nb, 5), jnp.float32)], axis=1)


def _make_encB0p(K, NBP):
    M = K * NBP

    def body(g_ref, t_ref, A1, T1, ONE3, wd_s, b1_s, fcWb, fcb_t,
             mWb, mb_t, W2b, b2_s, maskF, maskX, fagg_ref, fx2_ref):
        G = g_ref[...].reshape(M, 128)
        t = t_ref[...]
        tP = t @ T1[...]
        TB = jnp.broadcast_to(t[None], (K, NBP, 128)).reshape(M, 128)
        tPB = jnp.broadcast_to(tP[None], (K, NBP, 128)).reshape(M, 128)
        rel = (TB - G) * maskX[...]
        DIST = jnp.sqrt((rel * rel) @ ONE3[...] + 1e-12)
        FXs = jnp.maximum(
            DIST * wd_s[...] + G @ A1[...] + tPB + b1_s[...], 0.0)
        FSET = G * maskF[...] + FXs
        ATT = FSET @ fcWb[...] + fcb_t[...]
        A = ATT.reshape(K, NBP, 128)
        m = jnp.max(A, axis=0, keepdims=True)
        E = jnp.exp(A - m)
        S = jnp.sum(E, axis=0, keepdims=True)
        AGG = jnp.sum((E / S) * FSET.reshape(K, NBP, 128), axis=0)
        fagg_ref[...] = jnp.maximum(AGG @ mWb[...] + mb_t[...], 0.0)
        fx2_ref[...] = jnp.maximum(
            FXs @ W2b[...] + b2_s[...], 0.0).reshape(K, NBP, 128)

    return body


def _make_encC0p(K, NBP):
    M = K * NBP

    def body(g_ref, fx2_ref, fcWb, fcb_t, apWb, apb_t, maskF, fp_ref):
        FSET = (g_ref[...].reshape(M, 128) * maskF[...]
                + fx2_ref[...].reshape(M, 128))
        ATT = FSET @ fcWb[...] + fcb_t[...]
        A = ATT.reshape(K, NBP, 128)
        m = jnp.max(A, axis=0, keepdims=True)
        E = jnp.exp(A - m)
        S = jnp.sum(E, axis=0, keepdims=True)
        AGG = jnp.sum((E / S) * FSET.reshape(K, NBP, 128), axis=0)
        fp_ref[...] = jnp.maximum(AGG @ apWb[...] + apb_t[...], 0.0)

    return body


def _make_ktail(PAD):
    def body(fp_ref, f0_ref, m2W, m2b, scW, scb, out_ref):
        y = (fp_ref[...] @ m2W[...] + m2b[...]
             + f0_ref[...] @ scW[...] + scb[...])
        y = _leaky(y)
        if PAD:
            y = jnp.concatenate(
                [y, jnp.zeros((y.shape[0], PAD), jnp.float32)], axis=1)
        out_ref[...] = y
    return body


def _k4p_body(g_ref, xyz_ref, Wb, bt, PLACE, s_ref, t_ref):
    s = jnp.max(g_ref[...], axis=0)
    ft = jnp.maximum(s @ Wb[...] + bt[...], 0.0)
    t_ref[...] = ft + xyz_ref[...] @ PLACE[...]
    s_ref[...] = s


def _blk(P, m):
    return jnp.kron(jnp.eye(P, dtype=jnp.float32), m)


def _pp(P, v):
    return jnp.tile(v.reshape(1, 128 // P), (1, P))


def _encp_weights(lfa, P):
    C = 128 // P
    CF = C // 2
    DH = C - CF
    W1, b1 = _fold(lfa["mlp1"])
    wd = W1[0]
    Wr, Wt, Wn = W1[1:4], W1[4:7], W1[7:10]
    zC = jnp.zeros((C, C), jnp.float32)
    A1 = _blk(P, zC.at[CF:CF + 3, CF:].set(Wn - Wr))
    T1 = _blk(P, zC.at[CF:CF + 3, CF:].set(Wr + Wt))
    ONE3 = _blk(P, zC.at[CF:CF + 3, :].set(1.0))
    zH = jnp.zeros((CF,), jnp.float32)
    wd_s = _pp(P, jnp.concatenate([zH, wd]))
    b1_s = _pp(P, jnp.concatenate([zH, b1[0]]))
    fcWb = _blk(P, lfa["ap1"]["fcW"])
    fcb_t = _pp(P, lfa["ap1"]["fcb"])
    mW, mb = _fold(lfa["ap1"]["mlp"])
    mWb = _blk(P, zC.at[:, 0:DH].set(mW))
    mb_t = _pp(P, jnp.concatenate([mb[0], zH]))
    W2, b2 = _fold(lfa["mlp2"])
    W2b = _blk(P, zC.at[CF:, CF:].set(W2))
    b2_s = _pp(P, jnp.concatenate([zH, b2[0]]))
    maskF = _pp(P, jnp.concatenate([jnp.ones((CF,), jnp.float32), zH]))
    maskX = _pp(P, jnp.array(
        [0.0] * CF + [1.0] * 3 + [0.0] * (C - CF - 3), jnp.float32))
    return (A1, T1, ONE3, wd_s, b1_s, fcWb, fcb_t, mWb, mb_t, W2b, b2_s,
            maskF, maskX)


def _encp_cweights(lfa, P):
    C = 128 // P
    CF = C // 2
    apW, apb = _fold(lfa["ap2"]["mlp"])
    maskF = _pp(P, jnp.concatenate(
        [jnp.ones((CF,), jnp.float32), jnp.zeros((C - CF,), jnp.float32)]))
    return (_blk(P, lfa["ap2"]["fcW"]), _pp(P, lfa["ap2"]["fcb"]),
            _blk(P, apW), _pp(P, apb[0]), maskF)


def _k7_body(g_ref, W, b, out_ref):
    s = jnp.max(g_ref[...], axis=0)
    out_ref[...] = jnp.maximum(s @ W[...] + b[...], 0.0)


def _k8_body(a_ref, c_ref, W, b, out_ref):
    x = jnp.concatenate([a_ref[...], c_ref[...]], axis=1)
    out_ref[...] = jnp.maximum(x @ W[...] + b[...], 0.0)


def _k9_body(a_ref, c_ref, W1, b1, W2, b2, W3, b3, W4, b4, out_ref):
    x = jnp.concatenate([a_ref[...], c_ref[...]], axis=1)
    x = jnp.maximum(x @ W1[...] + b1[...], 0.0)
    x = jnp.maximum(x @ W2[...] + b2[...], 0.0)
    x = jnp.maximum(x @ W3[...] + b3[...], 0.0)
    out_ref[...] = x @ W4[...] + b4[...]


def _call(body, grid, in_specs, out_specs, out_shape, args):
    return pl.pallas_call(
        body, grid=grid, in_specs=in_specs, out_specs=out_specs,
        out_shape=out_shape)(*args)


def kernel(features, xyz0, xyz1, params, neigh_idx0, neigh_idx1,
           sub_idx0, sub_idx1, interp_idx0, interp_idx1):
    N = features.shape[1]
    K = neigh_idx0.shape[2]
    N1 = xyz1.shape[1]
    N2 = sub_idx1.shape[1]
    f32 = jnp.float32
    sds = jax.ShapeDtypeStruct

    x = features[0]
    xy0 = xyz0[0]
    xy1 = xyz1[0]
    nT0 = jnp.transpose(neigh_idx0[0]).reshape(-1).astype(jnp.int32)
    nT1 = jnp.transpose(neigh_idx1[0]).reshape(-1).astype(jnp.int32)
    sT0 = jnp.transpose(sub_idx0[0]).reshape(-1).astype(jnp.int32)
    sT1 = jnp.transpose(sub_idx1[0]).reshape(-1).astype(jnp.int32)
    ii0 = interp_idx0[0, :, 0].astype(jnp.int32)
    ii1 = interp_idx1[0, :, 0].astype(jnp.int32)

    p = params
    W0 = p["fc0"]["W"] * p["bn0"]["g"][None, :]
    b0 = (p["fc0"]["b"] * p["bn0"]["g"] + p["bn0"]["be"])[None, :]

    NB = 2048
    f0, t0 = _call(
        _k1_body, (N // NB,),
        [_rspec(NB, 6), _rspec(NB, 3), _wspec((6, 8)), _wspec((1, 8)),
         _wspec((8, 8)), _wspec((1, 8))],
        [_rspec(NB, 8), _rspec(NB, 16)],
        [sds((N, 8), f32), sds((N, 16), f32)],
        (x, xy0, W0, b0, *_fold(p["enc0"]["mlp1"])))

    l0 = p["enc0"]["lfa"]
    N8 = N // 8
    NBP = 128
    g1 = _gather_rows(t0, nT0, K).reshape(K, N8, 128)
    t0p = t0.reshape(N8, 128)
    w128 = _wspec((128, 128))
    v128 = _wspec((1, 128))
    fagg0p, fx2s = _call(
        _make_encB0p(K, NBP), (N8 // NBP,),
        [_kspec(K, NBP, 128), _rspec(NBP, 128),
         w128, w128, w128, v128, v128, w128, v128, w128, v128, w128, v128,
         v128, v128],
        [_rspec(NBP, 128), _kspec(K, NBP, 128)],
        [sds((N8, 128), f32), sds((K, N8, 128), f32)],
        (g1, t0p, *_encp_weights(l0, 8)))
    g2 = _gather_rows(fagg0p.reshape(N, 16), nT0, K).reshape(K, N8, 128)
    fp = _call(
        _make_encC0p(K, NBP), (N8 // NBP,),
        [_kspec(K, NBP, 128), _kspec(K, NBP, 128),
         w128, v128, w128, v128, v128],
        _rspec(NBP, 128),
        sds((N8, 128), f32),
        (g2, fx2s, *_encp_cweights(l0, 8)))
    NB = 2048
    e0 = _call(
        _make_ktail(32), (N // NB,),
        [_rspec(NB, 16), _rspec(NB, 8),
         _wspec((16, 32)), _wspec((1, 32)), _wspec((8, 32)), _wspec((1, 32))],
        _rspec(NB, 64),
        sds((N, 64), f32),
        (fp.reshape(N, 16), f0, *_fold(p["enc0"]["mlp2"]),
         *_fold(p["enc0"]["sc"])))

    N1h = N1 // 2
    gs0 = _gather_rows(e0, sT0, K).reshape(K, N1h, 128)
    W41, b41 = _fold(p["enc1"]["mlp1"])
    W4b = _blk(2, jnp.zeros((64, 64), jnp.float32).at[0:32, 0:32].set(W41))
    b4t = _pp(2, jnp.concatenate([b41[0], jnp.zeros((32,), jnp.float32)]))
    place = jnp.zeros((6, 128), jnp.float32)
    for _j in range(6):
        place = place.at[_j, (_j // 3) * 64 + 32 + (_j % 3)].set(1.0)
    xyz6 = xy1.reshape(N1h, 6)
    NBP4 = 128
    s0p, t1p = _call(
        _k4p_body, (N1h // NBP4,),
        [_kspec(K, NBP4, 128), _rspec(NBP4, 6),
         _wspec((128, 128)), _wspec((1, 128)), _wspec((6, 128))],
        [_rspec(NBP4, 128), _rspec(NBP4, 128)],
        [sds((N1h, 128), f32), sds((N1h, 128), f32)],
        (gs0, xyz6, W4b, b4t, place))
    t1 = t1p.reshape(N1, 64)
    s0 = s0p.reshape(N1, 64)

    l1 = p["enc1"]["lfa"]
    g1b = _gather_rows(t1, nT1, K).reshape(K, N1h, 128)
    t1p = t1.reshape(N1h, 128)
    fagg1p, fx2s1 = _call(
        _make_encB0p(K, NBP), (N1h // NBP,),
        [_kspec(K, NBP, 128), _rspec(NBP, 128),
         w128, w128, w128, v128, v128, w128, v128, w128, v128, w128, v128,
         v128, v128],
        [_rspec(NBP, 128), _kspec(K, NBP, 128)],
        [sds((N1h, 128), f32), sds((K, N1h, 128), f32)],
        (g1b, t1p, *_encp_weights(l1, 2)))
    g2b = _gather_rows(fagg1p.reshape(N1, 64), nT1, K).reshape(K, N1h, 128)
    fp1 = _call(
        _make_encC0p(K, NBP), (N1h // NBP,),
        [_kspec(K, NBP, 128), _kspec(K, NBP, 128),
         w128, v128, w128, v128, v128],
        _rspec(NBP, 128),
        sds((N1h, 128), f32),
        (g2b, fx2s1, *_encp_cweights(l1, 2)))
    NB = 2048
    scW1, scb1 = _fold(p["enc1"]["sc"])
    scW1p = jnp.concatenate([scW1, jnp.zeros((32, 128), jnp.float32)], axis=0)
    e1 = _call(
        _make_ktail(0), (N1 // NB,),
        [_rspec(NB, 64), _rspec(NB, 64),
         _wspec((64, 128)), _wspec((1, 128)),
         _wspec((64, 128)), _wspec((1, 128))],
        _rspec(NB, 128),
        sds((N1, 128), f32),
        (fp1.reshape(N1, 64), s0, *_fold(p["enc1"]["mlp2"]),
         scW1p, scb1))

    gs1 = _gather_rows(e1, sT1, K)
    NB = 512
    feat2 = _call(
        _k7_body, (N2 // NB,),
        [_kspec(K, NB, 128), _wspec((128, 128)), _wspec((1, 128))],
        _rspec(NB, 128),
        sds((N2, 128), f32),
        (gs1, *_fold(p["dec0_pre"])))

    fi1 = _gather_rows(feat2, ii1)
    NB = 2048
    W80, b80 = _fold(p["dec0"])
    W8p = jnp.concatenate(
        [W80[0:32], jnp.zeros((32, 32), jnp.float32), W80[32:]], axis=0)
    featd = _call(
        _k8_body, (N1 // NB,),
        [_rspec(NB, 64), _rspec(NB, 128), _wspec((192, 32)), _wspec((1, 32))],
        _rspec(NB, 32),
        sds((N1, 32), f32),
        (s0, fi1, W8p, b80))
    fi0 = _gather_rows(featd, ii0)
    NB = 2048
    W90, b90 = _fold(p["dec1"])
    W9p = jnp.concatenate(
        [W90[0:32], jnp.zeros((32, 32), jnp.float32), W90[32:]], axis=0)
    y = _call(
        _k9_body, (N // NB,),
        [_rspec(NB, 64), _rspec(NB, 32),
         _wspec((96, 32)), _wspec((1, 32)), _wspec((32, 64)), _wspec((1, 64)),
         _wspec((64, 32)), _wspec((1, 32)), _wspec((32, 13)), _wspec((1, 13))],
        _rspec(NB, 13),
        sds((N, 13), f32),
        (e0, fi0, W9p, b90, *_fold(p["fc1"]), *_fold(p["fc2"]),
         *_fold(p["fc"])))

    return jnp.transpose(y)[None]


# speedup vs baseline: 1.0149x; 1.0149x over previous
import functools

import jax
import jax.numpy as jnp
from jax import lax
from jax.experimental import pallas as pl
from jax.experimental.pallas import tpu as pltpu
from jax.experimental.pallas import tpu_sc as plsc

_NW = 32


@functools.lru_cache(maxsize=None)
def _make_gather(V, D, R, C):
    B = R * C
    bpw = B // _NW
    assert C % bpw == 0, (R, C, bpw)
    chunk = bpw
    while chunk * (D + 1) * 8 > 460000:
        chunk //= 2
    steps = bpw // chunk
    out_shape = (R, C, D) if R > 1 else (C, D)
    mesh = plsc.VectorSubcoreMesh(core_axis_name="c", subcore_axis_name="s")

    @functools.partial(
        pl.kernel, mesh=mesh,
        out_type=jax.ShapeDtypeStruct(out_shape, jnp.float32),
        scratch_types=[pltpu.VMEM((2, chunk), jnp.int32),
                       pltpu.VMEM((2, chunk, D), jnp.float32),
                       pltpu.SemaphoreType.DMA,
                       pltpu.SemaphoreType.DMA,
                       pltpu.SemaphoreType.DMA],
        compiler_params=pltpu.CompilerParams(use_tc_tiling_on_sc=False),
    )
    def gk(table_hbm, idx_hbm, out_hbm, idx_v, rows_v, sem_g, sem_i, sem_o):
        wid = lax.axis_index("s") * 2 + lax.axis_index("c")
        base = wid * bpw
        slab = base // C
        col0 = base % C

        def idx_cp(i, b):
            return pltpu.async_copy(
                idx_hbm.at[pl.ds(base + i * chunk, chunk)], idx_v.at[b],
                sem_i)

        def wb_cp(i, b):
            if R > 1:
                dst = out_hbm.at[slab, pl.ds(col0 + i * chunk, chunk)]
            else:
                dst = out_hbm.at[pl.ds(base + i * chunk, chunk)]
            return pltpu.async_copy(rows_v.at[b], dst, sem_o)

        cps = {}
        cps[("i", 0)] = idx_cp(0, 0)
        for i in range(steps):
            b = i % 2
            cps[("i", i)].wait()
            if i + 1 < steps:
                cps[("i", i + 1)] = idx_cp(i + 1, 1 - b)
            if i >= 2:
                cps[("o", i - 2)].wait()
            pltpu.async_copy(table_hbm.at[idx_v.at[b]], rows_v.at[b],
                             sem_g).wait()
            cps[("o", i)] = wb_cp(i, b)
        for i in range(max(0, steps - 2), steps):
            cps[("o", i)].wait()

    return gk


def _gather_rows(table, idx, R=1):
    V, D = table.shape
    (B,) = idx.shape
    return _make_gather(V, D, R, B // R)(table, idx)


def _wspec(shape):
    nd = len(shape)
    return pl.BlockSpec(shape, lambda i, _n=nd: (0,) * _n)


def _rspec(nb, c):
    return pl.BlockSpec((nb, c), lambda i: (i, 0))


def _kspec(k, nb, c):
    return pl.BlockSpec((k, nb, c), lambda i: (0, i, 0))


def _leaky(x):
    return jnp.where(x >= 0, x, 0.2 * x)


def _fold(p):
    return p["W"] * p["g"][None, :], (p["b"] * p["g"] + p["be"])[None, :]


def _k1_body(x_ref, xyz_ref, W0, b0, W1, b1, f0_ref, t0_ref):
    y = x_ref[...] @ W0[...] + b0[...]
    f0 = _leaky(y)
    ft = jnp.maximum(f0 @ W1[...] + b1[...], 0.0)
    f0_ref[...] = f0
    nb = ft.shape[0]
    t0_ref[...] = jnp.concatenate(
        [ft, xyz_ref[...], jnp.zeros((nb, 5), jnp.float32)], axis=1)


def _make_encB0p(K, NBP):
    M = K * NBP

    def body(g_ref, t_ref, A1, T1, ONE3, wd_s, b1_s, fcWb, fcb_t,
             mWb, mb_t, W2b, b2_s, maskF, maskX, fagg_ref, fx2_ref):
        G = g_ref[...].reshape(M, 128)
        t = t_ref[...]
        tP = t @ T1[...]
        TB = jnp.broadcast_to(t[None], (K, NBP, 128)).reshape(M, 128)
        tPB = jnp.broadcast_to(tP[None], (K, NBP, 128)).reshape(M, 128)
        rel = (TB - G) * maskX[...]
        DIST = jnp.sqrt((rel * rel) @ ONE3[...] + 1e-12)
        FXs = jnp.maximum(
            DIST * wd_s[...] + G @ A1[...] + tPB + b1_s[...], 0.0)
        FSET = G * maskF[...] + FXs
        ATT = FSET @ fcWb[...] + fcb_t[...]
        A = ATT.reshape(K, NBP, 128)
        m = jnp.max(A, axis=0, keepdims=True)
        E = jnp.exp(A - m)
        S = jnp.sum(E, axis=0, keepdims=True)
        AGG = jnp.sum((E / S) * FSET.reshape(K, NBP, 128), axis=0)
        fagg_ref[...] = jnp.maximum(AGG @ mWb[...] + mb_t[...], 0.0)
        fx2_ref[...] = jnp.maximum(
            FXs @ W2b[...] + b2_s[...], 0.0).reshape(K, NBP, 128)

    return body


def _make_encC0p(K, NBP):
    M = K * NBP

    def body(g_ref, fx2_ref, fcWb, fcb_t, apWb, apb_t, maskF, fp_ref):
        FSET = (g_ref[...].reshape(M, 128) * maskF[...]
                + fx2_ref[...].reshape(M, 128))
        ATT = FSET @ fcWb[...] + fcb_t[...]
        A = ATT.reshape(K, NBP, 128)
        m = jnp.max(A, axis=0, keepdims=True)
        E = jnp.exp(A - m)
        S = jnp.sum(E, axis=0, keepdims=True)
        AGG = jnp.sum((E / S) * FSET.reshape(K, NBP, 128), axis=0)
        fp_ref[...] = jnp.maximum(AGG @ apWb[...] + apb_t[...], 0.0)

    return body


def _make_ktail(PAD):
    def body(fp_ref, f0_ref, m2W, m2b, scW, scb, out_ref):
        y = (fp_ref[...] @ m2W[...] + m2b[...]
             + f0_ref[...] @ scW[...] + scb[...])
        y = _leaky(y)
        if PAD:
            y = jnp.concatenate(
                [y, jnp.zeros((y.shape[0], PAD), jnp.float32)], axis=1)
        out_ref[...] = y
    return body


def _k4p_body(g_ref, xyz_ref, Wb, bt, PLACE, s_ref, t_ref):
    s = jnp.max(g_ref[...], axis=0)
    ft = jnp.maximum(s @ Wb[...] + bt[...], 0.0)
    t_ref[...] = ft + xyz_ref[...] @ PLACE[...]
    s_ref[...] = s


def _blk(P, m):
    return jnp.kron(jnp.eye(P, dtype=jnp.float32), m)


def _pp(P, v):
    return jnp.tile(v.reshape(1, 128 // P), (1, P))


def _encp_weights(lfa, P):
    C = 128 // P
    CF = C // 2
    DH = C - CF
    W1, b1 = _fold(lfa["mlp1"])
    wd = W1[0]
    Wr, Wt, Wn = W1[1:4], W1[4:7], W1[7:10]
    zC = jnp.zeros((C, C), jnp.float32)
    A1 = _blk(P, zC.at[CF:CF + 3, CF:].set(Wn - Wr))
    T1 = _blk(P, zC.at[CF:CF + 3, CF:].set(Wr + Wt))
    ONE3 = _blk(P, zC.at[CF:CF + 3, :].set(1.0))
    zH = jnp.zeros((CF,), jnp.float32)
    wd_s = _pp(P, jnp.concatenate([zH, wd]))
    b1_s = _pp(P, jnp.concatenate([zH, b1[0]]))
    fcWb = _blk(P, lfa["ap1"]["fcW"])
    fcb_t = _pp(P, lfa["ap1"]["fcb"])
    mW, mb = _fold(lfa["ap1"]["mlp"])
    mWb = _blk(P, zC.at[:, 0:DH].set(mW))
    mb_t = _pp(P, jnp.concatenate([mb[0], zH]))
    W2, b2 = _fold(lfa["mlp2"])
    W2b = _blk(P, zC.at[CF:, CF:].set(W2))
    b2_s = _pp(P, jnp.concatenate([zH, b2[0]]))
    maskF = _pp(P, jnp.concatenate([jnp.ones((CF,), jnp.float32), zH]))
    maskX = _pp(P, jnp.array(
        [0.0] * CF + [1.0] * 3 + [0.0] * (C - CF - 3), jnp.float32))
    return (A1, T1, ONE3, wd_s, b1_s, fcWb, fcb_t, mWb, mb_t, W2b, b2_s,
            maskF, maskX)


def _encp_cweights(lfa, P):
    C = 128 // P
    CF = C // 2
    apW, apb = _fold(lfa["ap2"]["mlp"])
    maskF = _pp(P, jnp.concatenate(
        [jnp.ones((CF,), jnp.float32), jnp.zeros((C - CF,), jnp.float32)]))
    return (_blk(P, lfa["ap2"]["fcW"]), _pp(P, lfa["ap2"]["fcb"]),
            _blk(P, apW), _pp(P, apb[0]), maskF)


def _k7_body(g_ref, W, b, out_ref):
    s = jnp.max(g_ref[...], axis=0)
    out_ref[...] = jnp.maximum(s @ W[...] + b[...], 0.0)


def _k8_body(a_ref, c_ref, W, b, out_ref):
    x = jnp.concatenate([a_ref[...], c_ref[...]], axis=1)
    out_ref[...] = jnp.maximum(x @ W[...] + b[...], 0.0)


def _k9_body(a_ref, c_ref, W1, b1, W2, b2, W3, b3, W4, b4, out_ref):
    x = jnp.concatenate([a_ref[...], c_ref[...]], axis=1)
    x = jnp.maximum(x @ W1[...] + b1[...], 0.0)
    x = jnp.maximum(x @ W2[...] + b2[...], 0.0)
    x = jnp.maximum(x @ W3[...] + b3[...], 0.0)
    out_ref[...] = x @ W4[...] + b4[...]


def _call(body, grid, in_specs, out_specs, out_shape, args):
    return pl.pallas_call(
        body, grid=grid, in_specs=in_specs, out_specs=out_specs,
        out_shape=out_shape)(*args)


def kernel(features, xyz0, xyz1, params, neigh_idx0, neigh_idx1,
           sub_idx0, sub_idx1, interp_idx0, interp_idx1):
    N = features.shape[1]
    K = neigh_idx0.shape[2]
    N1 = xyz1.shape[1]
    N2 = sub_idx1.shape[1]
    f32 = jnp.float32
    sds = jax.ShapeDtypeStruct

    x = features[0]
    xy0 = xyz0[0]
    xy1 = xyz1[0]
    nT0 = jnp.transpose(neigh_idx0[0]).reshape(-1).astype(jnp.int32)
    nT1 = jnp.transpose(neigh_idx1[0]).reshape(-1).astype(jnp.int32)
    sT0 = jnp.transpose(sub_idx0[0]).reshape(-1).astype(jnp.int32)
    sT1 = jnp.transpose(sub_idx1[0]).reshape(-1).astype(jnp.int32)
    ii0 = interp_idx0[0, :, 0].astype(jnp.int32)
    ii1 = interp_idx1[0, :, 0].astype(jnp.int32)

    p = params
    W0 = p["fc0"]["W"] * p["bn0"]["g"][None, :]
    b0 = (p["fc0"]["b"] * p["bn0"]["g"] + p["bn0"]["be"])[None, :]

    NB = 2048
    f0, t0 = _call(
        _k1_body, (N // NB,),
        [_rspec(NB, 6), _rspec(NB, 3), _wspec((6, 8)), _wspec((1, 8)),
         _wspec((8, 8)), _wspec((1, 8))],
        [_rspec(NB, 8), _rspec(NB, 16)],
        [sds((N, 8), f32), sds((N, 16), f32)],
        (x, xy0, W0, b0, *_fold(p["enc0"]["mlp1"])))

    l0 = p["enc0"]["lfa"]
    N8 = N // 8
    NBP = 128
    g1 = _gather_rows(t0, nT0, K).reshape(K, N8, 128)
    t0p = t0.reshape(N8, 128)
    w128 = _wspec((128, 128))
    v128 = _wspec((1, 128))
    fagg0p, fx2s = _call(
        _make_encB0p(K, NBP), (N8 // NBP,),
        [_kspec(K, NBP, 128), _rspec(NBP, 128),
         w128, w128, w128, v128, v128, w128, v128, w128, v128, w128, v128,
         v128, v128],
        [_rspec(NBP, 128), _kspec(K, NBP, 128)],
        [sds((N8, 128), f32), sds((K, N8, 128), f32)],
        (g1, t0p, *_encp_weights(l0, 8)))
    g2 = _gather_rows(fagg0p.reshape(N, 16), nT0, K).reshape(K, N8, 128)
    fp = _call(
        _make_encC0p(K, NBP), (N8 // NBP,),
        [_kspec(K, NBP, 128), _kspec(K, NBP, 128),
         w128, v128, w128, v128, v128],
        _rspec(NBP, 128),
        sds((N8, 128), f32),
        (g2, fx2s, *_encp_cweights(l0, 8)))
    NB = 2048
    e0 = _call(
        _make_ktail(32), (N // NB,),
        [_rspec(NB, 16), _rspec(NB, 8),
         _wspec((16, 32)), _wspec((1, 32)), _wspec((8, 32)), _wspec((1, 32))],
        _rspec(NB, 64),
        sds((N, 64), f32),
        (fp.reshape(N, 16), f0, *_fold(p["enc0"]["mlp2"]),
         *_fold(p["enc0"]["sc"])))

    N1h = N1 // 2
    gs0 = _gather_rows(e0, sT0, K).reshape(K, N1h, 128)
    W41, b41 = _fold(p["enc1"]["mlp1"])
    W4b = _blk(2, jnp.zeros((64, 64), jnp.float32).at[0:32, 0:32].set(W41))
    b4t = _pp(2, jnp.concatenate([b41[0], jnp.zeros((32,), jnp.float32)]))
    place = jnp.zeros((6, 128), jnp.float32)
    for _j in range(6):
        place = place.at[_j, (_j // 3) * 64 + 32 + (_j % 3)].set(1.0)
    xyz6 = xy1.reshape(N1h, 6)
    NBP4 = 128
    s0p, t1p = _call(
        _k4p_body, (N1h // NBP4,),
        [_kspec(K, NBP4, 128), _rspec(NBP4, 6),
         _wspec((128, 128)), _wspec((1, 128)), _wspec((6, 128))],
        [_rspec(NBP4, 128), _rspec(NBP4, 128)],
        [sds((N1h, 128), f32), sds((N1h, 128), f32)],
        (gs0, xyz6, W4b, b4t, place))
    t1 = t1p.reshape(N1, 64)
    s0 = s0p.reshape(N1, 64)

    l1 = p["enc1"]["lfa"]
    g1b = _gather_rows(t1, nT1, K).reshape(K, N1h, 128)
    t1p = t1.reshape(N1h, 128)
    fagg1p, fx2s1 = _call(
        _make_encB0p(K, NBP), (N1h // NBP,),
        [_kspec(K, NBP, 128), _rspec(NBP, 128),
         w128, w128, w128, v128, v128, w128, v128, w128, v128, w128, v128,
         v128, v128],
        [_rspec(NBP, 128), _kspec(K, NBP, 128)],
        [sds((N1h, 128), f32), sds((K, N1h, 128), f32)],
        (g1b, t1p, *_encp_weights(l1, 2)))
    g2b = _gather_rows(fagg1p.reshape(N1, 64), nT1, K).reshape(K, N1h, 128)
    fp1 = _call(
        _make_encC0p(K, NBP), (N1h // NBP,),
        [_kspec(K, NBP, 128), _kspec(K, NBP, 128),
         w128, v128, w128, v128, v128],
        _rspec(NBP, 128),
        sds((N1h, 128), f32),
        (g2b, fx2s1, *_encp_cweights(l1, 2)))
    NB = 2048
    scW1, scb1 = _fold(p["enc1"]["sc"])
    scW1p = jnp.concatenate([scW1, jnp.zeros((32, 128), jnp.float32)], axis=0)
    e1 = _call(
        _make_ktail(0), (N1 // NB,),
        [_rspec(NB, 64), _rspec(NB, 64),
         _wspec((64, 128)), _wspec((1, 128)),
         _wspec((64, 128)), _wspec((1, 128))],
        _rspec(NB, 128),
        sds((N1, 128), f32),
        (fp1.reshape(N1, 64), s0, *_fold(p["enc1"]["mlp2"]),
         scW1p, scb1))

    gs1 = _gather_rows(e1, sT1, K)
    NB = 512
    feat2 = _call(
        _k7_body, (N2 // NB,),
        [_kspec(K, NB, 128), _wspec((128, 128)), _wspec((1, 128))],
        _rspec(NB, 128),
        sds((N2, 128), f32),
        (gs1, *_fold(p["dec0_pre"])))

    fi1 = _gather_rows(feat2, ii1)
    NB = 2048
    W80, b80 = _fold(p["dec0"])
    W8p = jnp.concatenate(
        [W80[0:32], jnp.zeros((32, 32), jnp.float32), W80[32:]], axis=0)
    featd = _call(
        _k8_body, (N1 // NB,),
        [_rspec(NB, 64), _rspec(NB, 128), _wspec((192, 32)), _wspec((1, 32))],
        _rspec(NB, 32),
        sds((N1, 32), f32),
        (s0, fi1, W8p, b80))
    fi0 = _gather_rows(featd, ii0)
    NB = 2048
    W90, b90 = _fold(p["dec1"])
    W9p = jnp.concatenate(
        [W90[0:32], jnp.zeros((32, 32), jnp.float32), W90[32:]], axis=0)
    y = _call(
        _k9_body, (N // NB,),
        [_rspec(NB, 64), _rspec(NB, 32),
         _wspec((96, 32)), _wspec((1, 32)), _wspec((32, 64)), _wspec((1, 64)),
         _wspec((64, 32)), _wspec((1, 32)), _wspec((32, 13)), _wspec((1, 13))],
        _rspec(NB, 13),
        sds((N, 13), f32),
        (e0, fi0, W9p, b90, *_fold(p["fc1"]), *_fold(p["fc2"]),
         *_fold(p["fc"])))

    return jnp.transpose(y)[None]
